# baseline (speedup 1.0000x reference)
import functools

import jax
import jax.numpy as jnp
from jax import lax
from jax.experimental import pallas as pl
from jax.experimental.pallas import tpu as pltpu
from jax.experimental.pallas import tpu_sc as plsc

_B, _V, _D = 4, 100000, 64
_N = 16384

_info = plsc.get_sparse_core_info()
_NC, _NS, _L = _info.num_cores, _info.num_subcores, _info.num_lanes
_NW = _NC * _NS
_ROWS_PER_W = (_B * _N) // _NW
_CHUNK = 1024
_NCHUNK = _ROWS_PER_W // _CHUNK


def _make_gather():
    mesh = plsc.VectorSubcoreMesh(core_axis_name="c", subcore_axis_name="s")

    @functools.partial(
        pl.kernel,
        mesh=mesh,
        out_type=jax.ShapeDtypeStruct((_B * _N, _D), jnp.float32),
        scratch_types=[
            pltpu.VMEM((_CHUNK,), jnp.int32),
            pltpu.VMEM((_CHUNK, _D), jnp.float32),
            pltpu.SemaphoreType.DMA,
        ],
        compiler_params=pltpu.CompilerParams(use_tc_tiling_on_sc=False),
    )
    def gather_kernel(table_hbm, idx_hbm, out_hbm, idx_v, rows_v, gsem):
        wid = lax.axis_index("s") * _NC + lax.axis_index("c")
        row_base = pl.multiple_of(wid * _ROWS_PER_W, _ROWS_PER_W)
        b = wid // (_N // _ROWS_PER_W)
        i_base = pl.multiple_of(row_base - b * _N, _CHUNK)
        boff = b * _V
        for c in range(_NCHUNK):
            pltpu.sync_copy(idx_hbm.at[pl.ds(i_base + c * _CHUNK, _CHUNK)], idx_v)
            for k in range(_CHUNK // _L):
                sl = pl.ds(k * _L, _L)
                idx_v[sl] = idx_v[sl] + boff
            pltpu.async_copy(table_hbm.at[idx_v], rows_v, gsem).wait()
            pltpu.sync_copy(rows_v, out_hbm.at[pl.ds(row_base + c * _CHUNK, _CHUNK)])

    return gather_kernel


_gather = _make_gather()


def kernel(input, indices):
    table = input.reshape(_B * _V, _D)
    idx = indices.astype(jnp.int32)
    out = _gather(table, idx)
    return out.reshape(_B, _N, _D)


# speedup vs baseline: 1.0020x; 1.0020x over previous
import functools

import jax
import jax.numpy as jnp
from jax import lax
from jax.experimental import pallas as pl
from jax.experimental.pallas import tpu as pltpu
from jax.experimental.pallas import tpu_sc as plsc

_B, _V, _D = 4, 100000, 64
_N = 16384

_info = plsc.get_sparse_core_info()
_NC, _NS, _L = _info.num_cores, _info.num_subcores, _info.num_lanes
_NW = _NC * _NS
_ROWS_PER_W = (_B * _N) // _NW
_CHUNK = 256
_NCHUNK = _ROWS_PER_W // _CHUNK
_NBUF = 4


def _make_gather():
    mesh = plsc.VectorSubcoreMesh(core_axis_name="c", subcore_axis_name="s")

    @functools.partial(
        pl.kernel,
        mesh=mesh,
        out_type=jax.ShapeDtypeStruct((_B * _N, _D), jnp.float32),
        scratch_types=[
            pltpu.VMEM((_ROWS_PER_W,), jnp.int32),
            pltpu.VMEM((_NBUF, _CHUNK, _D), jnp.float32),
            pltpu.SemaphoreType.DMA,
            pltpu.SemaphoreType.DMA,
        ],
        compiler_params=pltpu.CompilerParams(use_tc_tiling_on_sc=False),
    )
    def gather_kernel(table_hbm, idx_hbm, out_hbm, idx_v, bufs, gsem, ssem):
        wid = lax.axis_index("s") * _NC + lax.axis_index("c")
        row_base = pl.multiple_of(wid * _ROWS_PER_W, _ROWS_PER_W)
        b = wid // (_N // _ROWS_PER_W)
        i_base = pl.multiple_of(row_base - b * _N, _ROWS_PER_W)
        boff = b * _V

        pltpu.sync_copy(idx_hbm.at[pl.ds(i_base, _ROWS_PER_W)], idx_v)
        for k in range(_ROWS_PER_W // _L):
            sl = pl.ds(k * _L, _L)
            idx_v[sl] = idx_v[sl] + boff

        gathers = {}
        scatters = {}

        def gather_start(c):
            gathers[c] = pltpu.async_copy(
                table_hbm.at[idx_v.at[pl.ds(c * _CHUNK, _CHUNK)]],
                bufs.at[c % _NBUF],
                gsem,
            )

        def scatter_start(c):
            scatters[c] = pltpu.async_copy(
                bufs.at[c % _NBUF],
                out_hbm.at[pl.ds(row_base + c * _CHUNK, _CHUNK)],
                ssem,
            )

        gather_start(0)
        gather_start(1)
        drained = 0
        for c in range(_NCHUNK):
            gathers[c].wait()
            scatter_start(c)
            if c + 2 < _NCHUNK:
                if c >= 1:
                    scatters[drained].wait()
                    drained += 1
                gather_start(c + 2)
        for c in range(drained, _NCHUNK):
            scatters[c].wait()

    return gather_kernel


_gather = _make_gather()


def kernel(input, indices):
    table = input.reshape(_B * _V, _D)
    idx = indices.astype(jnp.int32)
    out = _gather(table, idx)
    return out.reshape(_B, _N, _D)


# speedup vs baseline: 2.0842x; 2.0800x over previous
import functools

import jax
import jax.numpy as jnp
from jax import lax
from jax.experimental import pallas as pl
from jax.experimental.pallas import tpu as pltpu
from jax.experimental.pallas import tpu_sc as plsc

_B, _V, _D = 4, 100000, 64
_N = 16384
_P = _B * _D

_info = plsc.get_sparse_core_info()
_NC, _NS, _L = _info.num_cores, _info.num_subcores, _info.num_lanes
_NW = _NC * _NS
_PW = _P // _NW
_OCH = 2048
_NOCH = _N // _OCH


def _make_gather():
    mesh = plsc.VectorSubcoreMesh(core_axis_name="c", subcore_axis_name="s")

    @functools.partial(
        pl.kernel,
        mesh=mesh,
        out_type=jax.ShapeDtypeStruct((_P, _N), jnp.float32),
        scratch_types=[
            pltpu.VMEM((_N,), jnp.int32),
            pltpu.VMEM((_V,), jnp.float32),
            pltpu.VMEM((2, _OCH), jnp.float32),
            pltpu.SemaphoreType.DMA,
        ],
        compiler_params=pltpu.CompilerParams(
            use_tc_tiling_on_sc=True, needs_layout_passes=False
        ),
    )
    def gather_kernel(tab_hbm, idx_hbm, out_hbm, idx_v, plane_v, obuf, wsem):
        wid = lax.axis_index("s") * _NC + lax.axis_index("c")
        pltpu.sync_copy(idx_hbm, idx_v)

        for q in range(_PW):
            p = wid * _PW + q
            pltpu.sync_copy(tab_hbm.at[p], plane_v)
            writes = {}
            for ch in range(_NOCH):
                s = ch % 2

                def body(j, _):
                    g = plsc.load_gather(
                        plane_v, [idx_v[pl.ds(ch * _OCH + j * _L, _L)]]
                    )
                    obuf[s, pl.ds(j * _L, _L)] = g
                    return ()

                if ch >= 2:
                    writes[ch - 2].wait()
                lax.fori_loop(0, _OCH // _L, body, (), unroll=8)
                writes[ch] = pltpu.async_copy(
                    obuf.at[s], out_hbm.at[p, pl.ds(ch * _OCH, _OCH)], wsem
                )
            writes[_NOCH - 2].wait()
            writes[_NOCH - 1].wait()

    return gather_kernel


_gather = _make_gather()


def kernel(input, indices):
    tab = jnp.transpose(input, (0, 2, 1)).reshape(_P, _V)
    idx = indices.astype(jnp.int32)
    out = _gather(tab, idx)
    return out.reshape(_B, _D, _N).transpose(0, 2, 1)
